# Initial kernel scaffold; baseline (speedup 1.0000x reference)
import functools

import jax
import jax.numpy as jnp
from jax import lax
from jax.experimental import pallas as pl
from jax.experimental.pallas import tpu as pltpu
from jax.experimental.pallas import tpu_sc as plsc

N = 50000
E = 800000
D_IN = 1433
H = 16
C = 7
K = 4

NC = 2
NS = 16
NW = NC * NS
B = 1024
EP = 819200
EW = EP // NW
NWIN = EW // B
CHUNK = 128
NCHUNK = B // CHUNK
CH = 1568
NP = NW * CH


def _coeff_body(p_ref, o_ref):
    u = p_ref[...]
    b00 = 1.0 - u[:, 0]
    b01 = u[:, 0]
    b10 = 1.0 - u[:, 1]
    b11 = u[:, 1]
    o_ref[...] = jnp.stack(
        [b00 * b10, b00 * b11, b01 * b10, b01 * b11], axis=1)


def _coeff(pseudo_p):
    blk = 8192
    grid = EP // blk
    return pl.pallas_call(
        _coeff_body,
        grid=(grid,),
        in_specs=[pl.BlockSpec((blk, 2), lambda i: (i, 0))],
        out_specs=pl.BlockSpec((blk, K), lambda i: (i, 0)),
        out_shape=jax.ShapeDtypeStruct((EP, K), jnp.float32),
    )(pseudo_p)


def _mm_body(x_ref, w_ref, o_ref):
    o_ref[...] = jnp.dot(x_ref[...], w_ref[...],
                         preferred_element_type=jnp.float32)


def _matmul1(x, w1f):
    bm = 1000
    grid = N // bm
    return pl.pallas_call(
        _mm_body,
        grid=(grid,),
        in_specs=[pl.BlockSpec((bm, D_IN), lambda i: (i, 0)),
                  pl.BlockSpec((D_IN, K * H), lambda i: (0, 0))],
        out_specs=pl.BlockSpec((bm, K * H), lambda i: (i, 0)),
        out_shape=jax.ShapeDtypeStruct((N, K * H), jnp.float32),
    )(x, w1f)


def _mid_body(a_ref, d_ref, w_ref, o_ref):
    a = a_ref[0] + a_ref[1]
    deg = jnp.maximum(d_ref[0] + d_ref[1], 1.0)
    h = a / deg[:, None]
    h = jnp.where(h > 0, h, jnp.exp(h) - 1.0)
    o_ref[...] = jnp.dot(h, w_ref[...], preferred_element_type=jnp.float32)


def _mid(acc, deg, w2f):
    bm = 3136
    grid = NP // bm
    return pl.pallas_call(
        _mid_body,
        grid=(grid,),
        in_specs=[pl.BlockSpec((NC, bm, H), lambda i: (0, i, 0)),
                  pl.BlockSpec((NC, bm), lambda i: (0, i)),
                  pl.BlockSpec((H, K * H), lambda i: (0, 0))],
        out_specs=pl.BlockSpec((bm, K * H), lambda i: (i, 0)),
        out_shape=jax.ShapeDtypeStruct((NP, K * H), jnp.float32),
    )(acc, deg, w2f)


def _final_body(a_ref, d_ref, o_ref):
    a = a_ref[0] + a_ref[1]
    deg = jnp.maximum(d_ref[0] + d_ref[1], 1.0)
    z = a[:, :C] / deg[:, None]
    m = jnp.max(z, axis=1, keepdims=True)
    z = z - m
    o_ref[...] = z - jnp.log(jnp.sum(jnp.exp(z), axis=1, keepdims=True))


def _final(acc, deg):
    bm = 2000
    grid = N // bm
    return pl.pallas_call(
        _final_body,
        grid=(grid,),
        in_specs=[pl.BlockSpec((NC, bm, H), lambda i: (0, i, 0)),
                  pl.BlockSpec((NC, bm), lambda i: (0, i))],
        out_specs=pl.BlockSpec((bm, C), lambda i: (i, 0)),
        out_shape=jax.ShapeDtypeStruct((N, C), jnp.float32),
    )(acc, deg)


def _sc_edge_pass(xw, src2d, dst2d, coeff, with_deg):
    mesh = plsc.VectorSubcoreMesh(
        core_axis_name="c", subcore_axis_name="s", num_cores=NC,
        num_subcores=NS)

    out_type = [jax.ShapeDtypeStruct((NC, NP, H), jnp.float32)]
    scratch = [
        pltpu.VMEM_SHARED((NP, H), jnp.float32),
        pltpu.VMEM((NCHUNK, CHUNK), jnp.int32),
        pltpu.VMEM((NCHUNK, CHUNK), jnp.int32),
        pltpu.VMEM((B, K), jnp.float32),
        pltpu.VMEM((B, K * H), jnp.float32),
        pltpu.VMEM((B, H), jnp.float32),
        pltpu.VMEM((CH, H), jnp.float32),
        pltpu.SemaphoreType.DMA,
    ]
    if with_deg:
        out_type.append(jax.ShapeDtypeStruct((NC, NP), jnp.float32))
        scratch += [
            pltpu.VMEM_SHARED((NP,), jnp.float32),
            pltpu.VMEM((CHUNK,), jnp.float32),
            pltpu.VMEM((CH,), jnp.float32),
        ]

    def body(xw_h, src_h, dst_h, co_h, acc_h, *rest):
        if with_deg:
            (deg_h, acc_sh, src_v, dst_v, co_v, rows_v, msg_v, z_v, sem,
             deg_sh, ones_v, dz_v) = rest
        else:
            (acc_sh, src_v, dst_v, co_v, rows_v, msg_v, z_v, sem) = rest
            deg_h = deg_sh = ones_v = dz_v = None

        cid = lax.axis_index("c")
        sid = lax.axis_index("s")
        wid = sid * NC + cid

        zeros16 = jnp.zeros((16,), jnp.float32)

        @functools.partial(plsc.parallel_loop, 0, CH)
        def _(i):
            z_v[i] = zeros16

        if with_deg:
            @functools.partial(plsc.parallel_loop, 0, CH // 16)
            def _(i):
                dz_v[pl.ds(i * 16, 16)] = zeros16

            @functools.partial(plsc.parallel_loop, 0, CHUNK // 16)
            def _(i):
                ones_v[pl.ds(i * 16, 16)] = jnp.ones((16,), jnp.float32)

        for half in range(2):
            off = (sid * 2 + half) * CH
            pltpu.sync_copy(z_v, acc_sh.at[pl.ds(off, CH)])
            if with_deg:
                pltpu.sync_copy(dz_v, deg_sh.at[pl.ds(off, CH)])
        plsc.subcore_barrier()

        def window(w, carry):
            ebase = wid * EW + w * B
            rbase = pl.multiple_of(ebase // CHUNK, NCHUNK)
            pltpu.sync_copy(src_h.at[pl.ds(rbase, NCHUNK)], src_v)
            pltpu.sync_copy(dst_h.at[pl.ds(rbase, NCHUNK)], dst_v)
            pltpu.sync_copy(co_h.at[pl.ds(pl.multiple_of(ebase, B), B)], co_v)

            descs = []
            for j in range(NCHUNK):
                descs.append(pltpu.async_copy(
                    xw_h.at[src_v.at[j]],
                    rows_v.at[pl.ds(j * CHUNK, CHUNK)], sem))
            for d in descs:
                d.wait()

            @functools.partial(plsc.parallel_loop, 0, B, unroll=8)
            def _(b):
                c0 = co_v[b, 0]
                c1 = co_v[b, 1]
                c2 = co_v[b, 2]
                c3 = co_v[b, 3]
                r0 = rows_v[b, pl.ds(0, H)]
                r1 = rows_v[b, pl.ds(H, H)]
                r2 = rows_v[b, pl.ds(2 * H, H)]
                r3 = rows_v[b, pl.ds(3 * H, H)]
                msg_v[b] = (c0 * r0 + c1 * r1) + (c2 * r2 + c3 * r3)

            for j in range(NCHUNK):
                pltpu.sync_copy(msg_v.at[pl.ds(j * CHUNK, CHUNK)],
                                acc_sh.at[dst_v.at[j]], add=True)
                if with_deg:
                    pltpu.sync_copy(ones_v, deg_sh.at[dst_v.at[j]], add=True)
            return carry

        lax.fori_loop(0, NWIN, window, 0)
        plsc.subcore_barrier()

        for half in range(2):
            off = (sid * 2 + half) * CH
            pltpu.sync_copy(acc_sh.at[pl.ds(off, CH)],
                            acc_h.at[cid, pl.ds(off, CH)])
            if with_deg:
                pltpu.sync_copy(deg_sh.at[pl.ds(off, CH)],
                                deg_h.at[cid, pl.ds(off, CH)])

    fn = pl.kernel(body, out_type=tuple(out_type), mesh=mesh,
                   scratch_types=tuple(scratch))
    return fn(xw, src2d, dst2d, coeff)


def kernel(x, edge_index, pseudo, W1, W2):
    src = edge_index[0]
    dst = edge_index[1]

    pad = EP - E
    pad_dst = N + (jnp.arange(pad, dtype=jnp.int32) % (NP - N))
    src_p = jnp.concatenate([src, jnp.zeros((pad,), jnp.int32)])
    dst_p = jnp.concatenate([dst, pad_dst])
    pseudo_p = jnp.concatenate([pseudo, jnp.zeros((pad, 2), jnp.float32)])

    src2d = src_p.reshape(EP // CHUNK, CHUNK)
    dst2d = dst_p.reshape(EP // CHUNK, CHUNK)

    coeff = _coeff(pseudo_p)

    w1f = W1.transpose(1, 0, 2).reshape(D_IN, K * H)
    xw1 = _matmul1(x, w1f)

    acc1, deg = _sc_edge_pass(xw1, src2d, dst2d, coeff, with_deg=True)

    w2p = jnp.pad(W2, ((0, 0), (0, 0), (0, H - C)))
    w2f = w2p.transpose(1, 0, 2).reshape(H, K * H)
    hw2 = _mid(acc1, deg, w2f)

    (acc2,) = _sc_edge_pass(hw2[:N], src2d, dst2d, coeff, with_deg=False)

    return _final(acc2, deg)


# speedup vs baseline: 5.4223x; 5.4223x over previous
import jax
import jax.numpy as jnp
from jax import lax
from jax.experimental import pallas as pl
from jax.experimental.pallas import tpu as pltpu
from jax.experimental.pallas import tpu_sc as plsc

N = 50000
E = 800000
D_IN = 1433
H = 16
C = 7
K = 4

NC = 2
NS = 16
NW = NC * NS
B = 256
EP = 802816
EW = EP // NW
NWIN = EW // B
CHUNK = 128
NCHUNK = B // CHUNK
CH = 1568
NP = NW * CH
ZB = 392
U = 8


def _coeff_body(p_ref, o_ref):
    u = p_ref[...]
    blk = u.shape[0]
    b00 = (1.0 - u[:, 0])[:, None]
    b01 = u[:, 0][:, None]
    b10 = (1.0 - u[:, 1])[:, None]
    b11 = u[:, 1][:, None]
    o_ref[...] = jnp.concatenate(
        [jnp.broadcast_to(b00 * b10, (blk, H)),
         jnp.broadcast_to(b00 * b11, (blk, H)),
         jnp.broadcast_to(b01 * b10, (blk, H)),
         jnp.broadcast_to(b01 * b11, (blk, H))], axis=1)


def _coeff(pseudo_p):
    blk = 8192
    grid = EP // blk
    return pl.pallas_call(
        _coeff_body,
        grid=(grid,),
        in_specs=[pl.BlockSpec((blk, 2), lambda i: (i, 0))],
        out_specs=pl.BlockSpec((blk, K * H), lambda i: (i, 0)),
        out_shape=jax.ShapeDtypeStruct((EP, K * H), jnp.float32),
    )(pseudo_p)


def _mm_body(x_ref, w_ref, o_ref):
    o_ref[...] = jnp.dot(x_ref[...], w_ref[...],
                         preferred_element_type=jnp.float32)


def _matmul1(x, w1f):
    bm = 1000
    grid = N // bm
    return pl.pallas_call(
        _mm_body,
        grid=(grid,),
        in_specs=[pl.BlockSpec((bm, D_IN), lambda i: (i, 0)),
                  pl.BlockSpec((D_IN, K * H), lambda i: (0, 0))],
        out_specs=pl.BlockSpec((bm, K * H), lambda i: (i, 0)),
        out_shape=jax.ShapeDtypeStruct((N, K * H), jnp.float32),
    )(x, w1f)


def _mid_body(a_ref, d_ref, w_ref, o_ref):
    a = a_ref[0] + a_ref[1]
    deg = jnp.maximum(d_ref[0] + d_ref[1], 1.0)
    h = a / deg
    h = jnp.where(h > 0, h, jnp.exp(h) - 1.0)
    o_ref[...] = jnp.dot(h, w_ref[...], preferred_element_type=jnp.float32)


def _mid(acc, deg3, w2f):
    bm = 3136
    grid = NP // bm
    return pl.pallas_call(
        _mid_body,
        grid=(grid,),
        in_specs=[pl.BlockSpec((NC, bm, H), lambda i: (0, i, 0)),
                  pl.BlockSpec((NC, bm, 1), lambda i: (0, i, 0)),
                  pl.BlockSpec((H, K * H), lambda i: (0, 0))],
        out_specs=pl.BlockSpec((bm, K * H), lambda i: (i, 0)),
        out_shape=jax.ShapeDtypeStruct((NP, K * H), jnp.float32),
    )(acc, deg3, w2f)


def _final_body(a_ref, d_ref, o_ref):
    a = a_ref[0] + a_ref[1]
    deg = jnp.maximum(d_ref[0] + d_ref[1], 1.0)
    z = a[:, :C] / deg
    m = jnp.max(z, axis=1, keepdims=True)
    z = z - m
    o_ref[...] = z - jnp.log(jnp.sum(jnp.exp(z), axis=1, keepdims=True))


def _final(acc, deg3):
    bm = 2000
    grid = N // bm
    return pl.pallas_call(
        _final_body,
        grid=(grid,),
        in_specs=[pl.BlockSpec((NC, bm, H), lambda i: (0, i, 0)),
                  pl.BlockSpec((NC, bm, 1), lambda i: (0, i, 0))],
        out_specs=pl.BlockSpec((bm, C), lambda i: (i, 0)),
        out_shape=jax.ShapeDtypeStruct((N, C), jnp.float32),
    )(acc, deg3)


def _sc_edge_pass(xw, src2d, dst2d, coeff, with_deg):
    mesh = plsc.VectorSubcoreMesh(
        core_axis_name="c", subcore_axis_name="s", num_cores=NC,
        num_subcores=NS)

    out_type = [jax.ShapeDtypeStruct((NC, NP, H), jnp.float32)]
    scratch = [
        pltpu.VMEM_SHARED((NP, H), jnp.float32),
        pltpu.VMEM((NCHUNK, CHUNK), jnp.int32),
        pltpu.VMEM((NCHUNK, CHUNK), jnp.int32),
        pltpu.VMEM((B, K * H), jnp.float32),
        pltpu.VMEM((B, K * H), jnp.float32),
        pltpu.VMEM((B, H), jnp.float32),
        pltpu.VMEM((ZB, H), jnp.float32),
        pltpu.SemaphoreType.DMA,
    ]
    if with_deg:
        out_type.append(jax.ShapeDtypeStruct((NC * NP,), jnp.float32))
        scratch += [
            pltpu.VMEM_SHARED((NP,), jnp.float32),
            pltpu.VMEM((CHUNK,), jnp.float32),
            pltpu.VMEM((ZB,), jnp.float32),
        ]

    def body(xw_h, src_h, dst_h, co_h, acc_h, *rest):
        if with_deg:
            (deg_h, acc_sh, src_v, dst_v, co_v, rows_v, msg_v, z_v, sem,
             deg_sh, ones_v, dz_v) = rest
        else:
            (acc_sh, src_v, dst_v, co_v, rows_v, msg_v, z_v, sem) = rest
            deg_h = deg_sh = ones_v = dz_v = None

        cid = lax.axis_index("c")
        sid = lax.axis_index("s")
        wid = sid * NC + cid

        zeros16 = jnp.zeros((16,), jnp.float32)
        ones16 = jnp.ones((16,), jnp.float32)

        def zrow(i, c):
            z_v[i] = zeros16
            return c
        lax.fori_loop(0, ZB, zrow, 0)

        if with_deg:
            def dzrow(i, c):
                off = jnp.minimum(i * 16, ZB - 16)
                dz_v[pl.ds(off, 16)] = zeros16
                return c
            lax.fori_loop(0, ZB // 16 + 1, dzrow, 0)

            def onerow(i, c):
                ones_v[pl.ds(i * 16, 16)] = ones16
                return c
            lax.fori_loop(0, CHUNK // 16, onerow, 0)

        for part in range(2 * CH // ZB):
            off = sid * 2 * CH + part * ZB
            pltpu.sync_copy(z_v, acc_sh.at[pl.ds(off, ZB)])
            if with_deg:
                pltpu.sync_copy(dz_v, deg_sh.at[pl.ds(off, ZB)])
        plsc.subcore_barrier()

        def window(w, carry):
            ebase = wid * EW + w * B
            rbase = pl.multiple_of(ebase // CHUNK, NCHUNK)
            pltpu.sync_copy(src_h.at[pl.ds(rbase, NCHUNK)], src_v)
            pltpu.sync_copy(dst_h.at[pl.ds(rbase, NCHUNK)], dst_v)
            pltpu.sync_copy(co_h.at[pl.ds(pl.multiple_of(ebase, B), B)], co_v)

            descs = []
            for j in range(NCHUNK):
                descs.append(pltpu.async_copy(
                    xw_h.at[src_v.at[j]],
                    rows_v.at[pl.ds(j * CHUNK, CHUNK)], sem))
            for d in descs:
                d.wait()

            def mixb(i, c):
                b0 = i * U
                for u in range(U):
                    b = b0 + u
                    p0 = rows_v[b, pl.ds(0, H)] * co_v[b, pl.ds(0, H)]
                    p1 = rows_v[b, pl.ds(H, H)] * co_v[b, pl.ds(H, H)]
                    p2 = (rows_v[b, pl.ds(2 * H, H)]
                          * co_v[b, pl.ds(2 * H, H)])
                    p3 = (rows_v[b, pl.ds(3 * H, H)]
                          * co_v[b, pl.ds(3 * H, H)])
                    msg_v[b] = (p0 + p1) + (p2 + p3)
                return c
            lax.fori_loop(0, B // U, mixb, 0)

            for j in range(NCHUNK):
                pltpu.sync_copy(msg_v.at[pl.ds(j * CHUNK, CHUNK)],
                                acc_sh.at[dst_v.at[j]], add=True)
                if with_deg:
                    pltpu.sync_copy(ones_v, deg_sh.at[dst_v.at[j]], add=True)
            return carry

        lax.fori_loop(0, NWIN, window, 0)
        plsc.subcore_barrier()

        for half in range(2):
            off = (sid * 2 + half) * CH
            pltpu.sync_copy(acc_sh.at[pl.ds(off, CH)],
                            acc_h.at[cid, pl.ds(off, CH)])
            if with_deg:
                doff = pl.multiple_of(cid * NP + off, 8)
                pltpu.sync_copy(deg_sh.at[pl.ds(off, CH)],
                                deg_h.at[pl.ds(doff, CH)])

    fn = pl.kernel(body, out_type=tuple(out_type), mesh=mesh,
                   scratch_types=tuple(scratch),
                   compiler_params=pltpu.CompilerParams(
                       use_tc_tiling_on_sc=False))
    return fn(xw, src2d, dst2d, coeff)


def kernel(x, edge_index, pseudo, W1, W2):
    src = edge_index[0]
    dst = edge_index[1]

    pad = EP - E
    pad_dst = N + (jnp.arange(pad, dtype=jnp.int32) % (NP - N))
    src_p = jnp.concatenate([src, jnp.zeros((pad,), jnp.int32)])
    dst_p = jnp.concatenate([dst, pad_dst])
    pseudo_p = jnp.concatenate([pseudo, jnp.zeros((pad, 2), jnp.float32)])

    src2d = src_p.reshape(EP // CHUNK, CHUNK)
    dst2d = dst_p.reshape(EP // CHUNK, CHUNK)

    coeff = _coeff(pseudo_p)

    w1f = W1.transpose(1, 0, 2).reshape(D_IN, K * H)
    xw1 = _matmul1(x, w1f)

    acc1, deg = _sc_edge_pass(xw1, src2d, dst2d, coeff, with_deg=True)
    deg3 = deg.reshape(NC, NP, 1)

    w2p = jnp.pad(W2, ((0, 0), (0, 0), (0, H - C)))
    w2f = w2p.transpose(1, 0, 2).reshape(H, K * H)
    hw2 = _mid(acc1, deg3, w2f)

    acc2 = _sc_edge_pass(hw2, src2d, dst2d, coeff, with_deg=False)
    if isinstance(acc2, (tuple, list)):
        acc2 = acc2[0]

    return _final(acc2, deg3)


# speedup vs baseline: 5.5492x; 1.0234x over previous
import jax
import jax.numpy as jnp
from jax import lax
from jax.experimental import pallas as pl
from jax.experimental.pallas import tpu as pltpu
from jax.experimental.pallas import tpu_sc as plsc

N = 50000
E = 800000
D_IN = 1433
H = 16
C = 7
K = 4

NC = 2
NS = 16
NW = NC * NS
B = 256
EP = 802816
EW = EP // NW
NWIN = EW // B
CHUNK = 128
NCHUNK = B // CHUNK
CH = 1568
NP = NW * CH
ZB = 392
U = 8


def _coeff_body(p_ref, o_ref):
    u = p_ref[...]
    blk2 = u.shape[0]
    outs = []
    for e in range(2):
        b00 = (1.0 - u[:, 2 * e])[:, None]
        b01 = u[:, 2 * e][:, None]
        b10 = (1.0 - u[:, 2 * e + 1])[:, None]
        b11 = u[:, 2 * e + 1][:, None]
        outs += [jnp.broadcast_to(b00 * b10, (blk2, H)),
                 jnp.broadcast_to(b00 * b11, (blk2, H)),
                 jnp.broadcast_to(b01 * b10, (blk2, H)),
                 jnp.broadcast_to(b01 * b11, (blk2, H))]
    o_ref[...] = jnp.concatenate(outs, axis=1)


def _coeff(pseudo_p):
    blk2 = 4096
    grid = (EP // 2) // blk2
    return pl.pallas_call(
        _coeff_body,
        grid=(grid,),
        in_specs=[pl.BlockSpec((blk2, 4), lambda i: (i, 0))],
        out_specs=pl.BlockSpec((blk2, 2 * K * H), lambda i: (i, 0)),
        out_shape=jax.ShapeDtypeStruct((EP // 2, 2 * K * H), jnp.float32),
    )(pseudo_p)


def _mm_body(x_ref, w_ref, o_ref):
    o_ref[...] = jnp.dot(x_ref[...], w_ref[...],
                         preferred_element_type=jnp.float32)


def _matmul1(x, w1f):
    bm = 1000
    grid = N // bm
    return pl.pallas_call(
        _mm_body,
        grid=(grid,),
        in_specs=[pl.BlockSpec((bm, D_IN), lambda i: (i, 0)),
                  pl.BlockSpec((D_IN, K * H), lambda i: (0, 0))],
        out_specs=pl.BlockSpec((bm, K * H), lambda i: (i, 0)),
        out_shape=jax.ShapeDtypeStruct((N, K * H), jnp.float32),
    )(x, w1f)


def _mid_body(a_ref, d_ref, w_ref, o_ref):
    a = a_ref[0] + a_ref[1]
    deg = jnp.maximum(d_ref[0] + d_ref[1], 1.0)
    h = a / deg
    h = jnp.where(h > 0, h, jnp.exp(h) - 1.0)
    o_ref[...] = jnp.dot(h, w_ref[...], preferred_element_type=jnp.float32)


def _mid(acc, deg3, w2f):
    bm = 3136
    grid = NP // bm
    return pl.pallas_call(
        _mid_body,
        grid=(grid,),
        in_specs=[pl.BlockSpec((NC, bm, H), lambda i: (0, i, 0)),
                  pl.BlockSpec((NC, bm, 1), lambda i: (0, i, 0)),
                  pl.BlockSpec((H, K * H), lambda i: (0, 0))],
        out_specs=pl.BlockSpec((bm, K * H), lambda i: (i, 0)),
        out_shape=jax.ShapeDtypeStruct((NP, K * H), jnp.float32),
    )(acc, deg3, w2f)


def _final_body(a_ref, d_ref, o_ref):
    a = a_ref[0] + a_ref[1]
    deg = jnp.maximum(d_ref[0] + d_ref[1], 1.0)
    z = a[:, :C] / deg
    m = jnp.max(z, axis=1, keepdims=True)
    z = z - m
    o_ref[...] = z - jnp.log(jnp.sum(jnp.exp(z), axis=1, keepdims=True))


def _final(acc, deg3):
    bm = 2000
    grid = N // bm
    return pl.pallas_call(
        _final_body,
        grid=(grid,),
        in_specs=[pl.BlockSpec((NC, bm, H), lambda i: (0, i, 0)),
                  pl.BlockSpec((NC, bm, 1), lambda i: (0, i, 0))],
        out_specs=pl.BlockSpec((bm, C), lambda i: (i, 0)),
        out_shape=jax.ShapeDtypeStruct((N, C), jnp.float32),
    )(acc, deg3)


def _sc_edge_pass(xw, src2d, dst2d, coeff, with_deg):
    mesh = plsc.VectorSubcoreMesh(
        core_axis_name="c", subcore_axis_name="s", num_cores=NC,
        num_subcores=NS)

    out_type = [jax.ShapeDtypeStruct((NC, NP, H), jnp.float32)]
    scratch = [
        pltpu.VMEM_SHARED((NP, H), jnp.float32),
        pltpu.VMEM((NCHUNK, CHUNK), jnp.int32),
        pltpu.VMEM((NCHUNK, CHUNK), jnp.int32),
        pltpu.VMEM((B // 2, 2 * K * H), jnp.float32),
        pltpu.VMEM((B, K * H), jnp.float32),
        pltpu.VMEM((B, H), jnp.float32),
        pltpu.VMEM((ZB, H), jnp.float32),
        pltpu.SemaphoreType.DMA,
    ]
    if with_deg:
        out_type.append(jax.ShapeDtypeStruct((NC * NP,), jnp.float32))
        scratch += [
            pltpu.VMEM_SHARED((NP,), jnp.float32),
            pltpu.VMEM((CHUNK,), jnp.float32),
            pltpu.VMEM((ZB,), jnp.float32),
        ]

    def body(xw_h, src_h, dst_h, co_h, acc_h, *rest):
        if with_deg:
            (deg_h, acc_sh, src_v, dst_v, co_v, rows_v, msg_v, z_v, sem,
             deg_sh, ones_v, dz_v) = rest
        else:
            (acc_sh, src_v, dst_v, co_v, rows_v, msg_v, z_v, sem) = rest
            deg_h = deg_sh = ones_v = dz_v = None

        cid = lax.axis_index("c")
        sid = lax.axis_index("s")
        wid = sid * NC + cid

        zeros16 = jnp.zeros((16,), jnp.float32)
        ones16 = jnp.ones((16,), jnp.float32)

        def zrow(i, c):
            z_v[i] = zeros16
            return c
        lax.fori_loop(0, ZB, zrow, 0)

        if with_deg:
            def dzrow(i, c):
                off = jnp.minimum(i * 16, ZB - 16)
                dz_v[pl.ds(off, 16)] = zeros16
                return c
            lax.fori_loop(0, ZB // 16 + 1, dzrow, 0)

            def onerow(i, c):
                ones_v[pl.ds(i * 16, 16)] = ones16
                return c
            lax.fori_loop(0, CHUNK // 16, onerow, 0)

        for part in range(2 * CH // ZB):
            off = sid * 2 * CH + part * ZB
            pltpu.sync_copy(z_v, acc_sh.at[pl.ds(off, ZB)])
            if with_deg:
                pltpu.sync_copy(dz_v, deg_sh.at[pl.ds(off, ZB)])
        plsc.subcore_barrier()

        def window(w, carry):
            ebase = wid * EW + w * B
            rbase = pl.multiple_of(ebase // CHUNK, NCHUNK)
            pltpu.sync_copy(src_h.at[pl.ds(rbase, NCHUNK)], src_v)
            pltpu.sync_copy(dst_h.at[pl.ds(rbase, NCHUNK)], dst_v)
            pltpu.sync_copy(
                co_h.at[pl.ds(pl.multiple_of(ebase // 2, B // 2), B // 2)],
                co_v)

            descs = []
            for j in range(NCHUNK):
                descs.append(pltpu.async_copy(
                    xw_h.at[src_v.at[j]],
                    rows_v.at[pl.ds(j * CHUNK, CHUNK)], sem))
            for d in descs:
                d.wait()

            def mixb(i, c):
                i0 = i * (U // 2)
                for u in range(U // 2):
                    ii = i0 + u
                    for p in range(2):
                        b = 2 * ii + p
                        cb = p * K * H
                        p0 = (rows_v[b, pl.ds(0, H)]
                              * co_v[ii, pl.ds(cb, H)])
                        p1 = (rows_v[b, pl.ds(H, H)]
                              * co_v[ii, pl.ds(cb + H, H)])
                        p2 = (rows_v[b, pl.ds(2 * H, H)]
                              * co_v[ii, pl.ds(cb + 2 * H, H)])
                        p3 = (rows_v[b, pl.ds(3 * H, H)]
                              * co_v[ii, pl.ds(cb + 3 * H, H)])
                        msg_v[b] = (p0 + p1) + (p2 + p3)
                return c
            lax.fori_loop(0, B // U, mixb, 0)

            for j in range(NCHUNK):
                pltpu.sync_copy(msg_v.at[pl.ds(j * CHUNK, CHUNK)],
                                acc_sh.at[dst_v.at[j]], add=True)
                if with_deg:
                    pltpu.sync_copy(ones_v, deg_sh.at[dst_v.at[j]], add=True)
            return carry

        lax.fori_loop(0, NWIN, window, 0)
        plsc.subcore_barrier()

        for half in range(2):
            off = (sid * 2 + half) * CH
            pltpu.sync_copy(acc_sh.at[pl.ds(off, CH)],
                            acc_h.at[cid, pl.ds(off, CH)])
            if with_deg:
                doff = pl.multiple_of(cid * NP + off, 8)
                pltpu.sync_copy(deg_sh.at[pl.ds(off, CH)],
                                deg_h.at[pl.ds(doff, CH)])

    fn = pl.kernel(body, out_type=tuple(out_type), mesh=mesh,
                   scratch_types=tuple(scratch),
                   compiler_params=pltpu.CompilerParams(
                       use_tc_tiling_on_sc=False))
    return fn(xw, src2d, dst2d, coeff)


def kernel(x, edge_index, pseudo, W1, W2):
    src = edge_index[0]
    dst = edge_index[1]

    pad = EP - E
    pad_dst = N + (jnp.arange(pad, dtype=jnp.int32) % (NP - N))
    src_p = jnp.concatenate([src, jnp.zeros((pad,), jnp.int32)])
    dst_p = jnp.concatenate([dst, pad_dst])
    pseudo_p = jnp.concatenate([pseudo, jnp.zeros((pad, 2), jnp.float32)])

    src2d = src_p.reshape(EP // CHUNK, CHUNK)
    dst2d = dst_p.reshape(EP // CHUNK, CHUNK)

    coeff = _coeff(pseudo_p.reshape(EP // 2, 4))

    w1f = W1.transpose(1, 0, 2).reshape(D_IN, K * H)
    xw1 = _matmul1(x, w1f)

    acc1, deg = _sc_edge_pass(xw1, src2d, dst2d, coeff, with_deg=True)
    deg3 = deg.reshape(NC, NP, 1)

    w2p = jnp.pad(W2, ((0, 0), (0, 0), (0, H - C)))
    w2f = w2p.transpose(1, 0, 2).reshape(H, K * H)
    hw2 = _mid(acc1, deg3, w2f)

    acc2 = _sc_edge_pass(hw2, src2d, dst2d, coeff, with_deg=False)
    if isinstance(acc2, (tuple, list)):
        acc2 = acc2[0]

    return _final(acc2, deg3)
